# speedup vs baseline: 2.1829x; 2.1829x over previous
import functools

import jax
import jax.numpy as jnp
from jax import lax
from jax.experimental import pallas as pl
from jax.experimental.pallas import tpu as pltpu
from jax.experimental.pallas import tpu_sc as plsc

N_ATOMS = 10000
N_EDGES = 320000
D = 128
N_RBF = 20

_LN2 = 0.6931471805599453

NC, NS = 2, 16
NW = NC * NS
EPW = N_EDGES // NW
CHUNK = 80
NCHUNK = EPW // CHUNK
P = 10240
RPS = P // NS


def _ssp(v):
    return jnp.maximum(v, 0.0) + jnp.log1p(jnp.exp(-jnp.abs(v))) - _LN2


def _h_body(x_ref, w_ref, b_ref, o_ref):
    o_ref[...] = (
        jnp.dot(x_ref[...], w_ref[...], preferred_element_type=jnp.float32)
        + b_ref[...]
    )


def _compute_h(x, W, b):
    blk = 2000
    return pl.pallas_call(
        _h_body,
        grid=(N_ATOMS // blk,),
        in_specs=[
            pl.BlockSpec((blk, D), lambda i: (i, 0)),
            pl.BlockSpec((D, D), lambda i: (0, 0)),
            pl.BlockSpec((1, D), lambda i: (0, 0)),
        ],
        out_specs=pl.BlockSpec((blk, D), lambda i: (i, 0)),
        out_shape=jax.ShapeDtypeStruct((N_ATOMS, D), jnp.float32),
    )(x, W, b.reshape(1, D))


def _wij_body(f_ref, rc_ref, w1_ref, b1_ref, w2_ref, b2_ref, o_ref):
    t = _ssp(
        jnp.dot(f_ref[...], w1_ref[...], preferred_element_type=jnp.float32)
        + b1_ref[...]
    )
    wij = (
        jnp.dot(t, w2_ref[...], preferred_element_type=jnp.float32)
        + b2_ref[...]
    )
    o_ref[...] = wij * rc_ref[...]


def _compute_wij(f_ij, rcut, W1, b1, W2, b2):
    blk = 4000
    return pl.pallas_call(
        _wij_body,
        grid=(N_EDGES // blk,),
        in_specs=[
            pl.BlockSpec((blk, N_RBF), lambda i: (i, 0)),
            pl.BlockSpec((blk, 1), lambda i: (i, 0)),
            pl.BlockSpec((N_RBF, D), lambda i: (0, 0)),
            pl.BlockSpec((1, D), lambda i: (0, 0)),
            pl.BlockSpec((D, D), lambda i: (0, 0)),
            pl.BlockSpec((1, D), lambda i: (0, 0)),
        ],
        out_specs=pl.BlockSpec((blk, D), lambda i: (i, 0)),
        out_shape=jax.ShapeDtypeStruct((N_EDGES, D), jnp.float32),
    )(f_ij, rcut.reshape(N_EDGES, 1), W1, b1.reshape(1, D), W2, b2.reshape(1, D))


def _sc_conv_body(h_hbm, wij_hbm, idx_i_hbm, idx_j_hbm, out_hbm,
                  idxj_v, idxi_v, rows_v, w_v, agg_s, sem):
    c = lax.axis_index("c")
    s = lax.axis_index("s")
    wid = s * NC + c
    base = wid * EPW

    zero = jnp.zeros((16,), jnp.float32)

    def zero_row(r, carry):
        for g in range(D // 16):
            rows_v[r, pl.ds(g * 16, 16)] = zero
        return carry

    lax.fori_loop(0, CHUNK, zero_row, 0)
    for k in range(RPS // CHUNK):
        pltpu.sync_copy(rows_v, agg_s.at[pl.ds(s * RPS + k * CHUNK, CHUNK)])
    plsc.subcore_barrier()

    def chunk_body(j, carry):
        eb = base + j * CHUNK
        pltpu.sync_copy(idx_j_hbm.at[pl.ds(eb, CHUNK)], idxj_v)
        pltpu.sync_copy(idx_i_hbm.at[pl.ds(eb, CHUNK)], idxi_v)
        gat = pltpu.async_copy(h_hbm.at[idxj_v], rows_v, sem)
        pltpu.sync_copy(wij_hbm.at[pl.ds(eb, CHUNK)], w_v)
        gat.wait()

        def mul_row(r, carry2):
            for g in range(D // 16):
                sl = pl.ds(g * 16, 16)
                rows_v[r, sl] = rows_v[r, sl] * w_v[r, sl]
            return carry2

        lax.fori_loop(0, CHUNK, mul_row, 0)
        pltpu.sync_copy(rows_v, agg_s.at[idxi_v], add=True)
        return carry

    lax.fori_loop(0, NCHUNK, chunk_body, 0)

    plsc.subcore_barrier()
    pltpu.sync_copy(
        agg_s.at[pl.ds(s * RPS, RPS)],
        out_hbm.at[c, pl.ds(s * RPS, RPS)],
    )


def _sc_conv(h, wij, idx_i, idx_j):
    mesh = plsc.VectorSubcoreMesh(core_axis_name="c", subcore_axis_name="s")
    run = pl.kernel(
        _sc_conv_body,
        out_type=jax.ShapeDtypeStruct((NC, P, D), jnp.float32),
        mesh=mesh,
        scratch_types=[
            pltpu.VMEM((CHUNK,), jnp.int32),
            pltpu.VMEM((CHUNK,), jnp.int32),
            pltpu.VMEM((CHUNK, D), jnp.float32),
            pltpu.VMEM((CHUNK, D), jnp.float32),
            pltpu.VMEM_SHARED((P, D), jnp.float32),
            pltpu.SemaphoreType.DMA,
        ],
    )
    return run(h, wij, idx_i, idx_j)


def _out_body(agg_ref, w1_ref, b1_ref, w2_ref, b2_ref, o_ref):
    a = agg_ref[0] + agg_ref[1]
    t = _ssp(
        jnp.dot(a, w1_ref[...], preferred_element_type=jnp.float32)
        + b1_ref[...]
    )
    o_ref[...] = (
        jnp.dot(t, w2_ref[...], preferred_element_type=jnp.float32)
        + b2_ref[...]
    )


def _compute_out(aggP, W1, b1, W2, b2):
    blk = 2000
    return pl.pallas_call(
        _out_body,
        grid=(N_ATOMS // blk,),
        in_specs=[
            pl.BlockSpec((NC, blk, D), lambda i: (0, i, 0)),
            pl.BlockSpec((D, D), lambda i: (0, 0)),
            pl.BlockSpec((1, D), lambda i: (0, 0)),
            pl.BlockSpec((D, D), lambda i: (0, 0)),
            pl.BlockSpec((1, D), lambda i: (0, 0)),
        ],
        out_specs=pl.BlockSpec((blk, D), lambda i: (i, 0)),
        out_shape=jax.ShapeDtypeStruct((N_ATOMS, D), jnp.float32),
    )(aggP, W1, b1.reshape(1, D), W2, b2.reshape(1, D))


@jax.jit
def kernel(x, f_ij, idx_i, idx_j, rcut_ij,
           W_in2f, b_in2f, W_f1, b_f1, W_f2, b_f2,
           W_o1, b_o1, W_o2, b_o2):
    idx_i = idx_i.astype(jnp.int32)
    idx_j = idx_j.astype(jnp.int32)
    h = _compute_h(x, W_in2f, b_in2f)
    wij = _compute_wij(f_ij, rcut_ij, W_f1, b_f1, W_f2, b_f2)
    aggP = _sc_conv(h, wij, idx_i, idx_j)
    return _compute_out(aggP, W_o1, b_o1, W_o2, b_o2)


# speedup vs baseline: 2.8403x; 1.3012x over previous
import functools

import jax
import jax.numpy as jnp
from jax import lax
from jax.experimental import pallas as pl
from jax.experimental.pallas import tpu as pltpu
from jax.experimental.pallas import tpu_sc as plsc

N_ATOMS = 10000
N_EDGES = 320000
D = 128
N_RBF = 20

_LN2 = 0.6931471805599453

NC, NS = 2, 16
NW = NC * NS
EPW = N_EDGES // NW
CHUNK = 40
NCHUNK = EPW // CHUNK
P = 10240
RPS = P // NS


def _ssp(v):
    return jnp.maximum(v, 0.0) + jnp.log1p(jnp.exp(-jnp.abs(v))) - _LN2


def _h_body(x_ref, w_ref, b_ref, o_ref):
    o_ref[...] = (
        jnp.dot(x_ref[...], w_ref[...], preferred_element_type=jnp.float32)
        + b_ref[...]
    )


def _compute_h(x, W, b):
    blk = 2000
    return pl.pallas_call(
        _h_body,
        grid=(N_ATOMS // blk,),
        in_specs=[
            pl.BlockSpec((blk, D), lambda i: (i, 0)),
            pl.BlockSpec((D, D), lambda i: (0, 0)),
            pl.BlockSpec((1, D), lambda i: (0, 0)),
        ],
        out_specs=pl.BlockSpec((blk, D), lambda i: (i, 0)),
        out_shape=jax.ShapeDtypeStruct((N_ATOMS, D), jnp.float32),
    )(x, W, b.reshape(1, D))


def _wij_body(f_ref, rc_ref, w1_ref, b1_ref, w2_ref, b2_ref, o_ref):
    t = _ssp(
        jnp.dot(f_ref[...], w1_ref[...], preferred_element_type=jnp.float32)
        + b1_ref[...]
    )
    wij = (
        jnp.dot(t, w2_ref[...], preferred_element_type=jnp.float32)
        + b2_ref[...]
    )
    o_ref[...] = wij * rc_ref[...]


def _compute_wij(f_ij, rcut, W1, b1, W2, b2):
    blk = 2560
    return pl.pallas_call(
        _wij_body,
        grid=(N_EDGES // blk,),
        in_specs=[
            pl.BlockSpec((blk, N_RBF), lambda i: (i, 0)),
            pl.BlockSpec((blk, 1), lambda i: (i, 0)),
            pl.BlockSpec((N_RBF, D), lambda i: (0, 0)),
            pl.BlockSpec((1, D), lambda i: (0, 0)),
            pl.BlockSpec((D, D), lambda i: (0, 0)),
            pl.BlockSpec((1, D), lambda i: (0, 0)),
        ],
        out_specs=pl.BlockSpec((blk, D), lambda i: (i, 0)),
        out_shape=jax.ShapeDtypeStruct((N_EDGES, D), jnp.float32),
    )(f_ij, rcut.reshape(N_EDGES, 1), W1, b1.reshape(1, D),
      W2, b2.reshape(1, D))


def _sc_conv_body(h_hbm, wij_hbm, idx_i_hbm, idx_j_hbm, out_hbm,
                  idxj_all,
                  rows0, rows1, w0, w1, out0, out1,
                  idxi0, idxi1, idxi2, idxi3,
                  agg_s,
                  gsem0, gsem1, wsem0, wsem1,
                  isem0, isem1, isem2, isem3, ssem0, ssem1):
    c = lax.axis_index("c")
    s = lax.axis_index("s")
    wid = s * NC + c
    base = wid * EPW

    rows = (rows0, rows1)
    w = (w0, w1)
    outb = (out0, out1)
    idxi = (idxi0, idxi1, idxi2, idxi3)
    gsem = (gsem0, gsem1)
    wsem = (wsem0, wsem1)
    isem = (isem0, isem1, isem2, isem3)
    ssem = (ssem0, ssem1)

    zero = jnp.zeros((16,), jnp.float32)

    @plsc.parallel_loop(0, CHUNK, 1, unroll=4)
    def _(r):
        for g in range(D // 16):
            out0[r, pl.ds(g * 16, 16)] = zero

    for k in range(RPS // CHUNK):
        pltpu.sync_copy(out0, agg_s.at[pl.ds(s * RPS + k * CHUNK, CHUNK)])
    plsc.subcore_barrier()

    pltpu.sync_copy(idx_j_hbm.at[pl.ds(base, EPW)], idxj_all)

    def issue_loads(j, p, q):
        eb = base + j * CHUNK
        pltpu.async_copy(
            h_hbm.at[idxj_all.at[pl.ds(j * CHUNK, CHUNK)]], rows[p], gsem[p])
        pltpu.async_copy(wij_hbm.at[pl.ds(eb, CHUNK)], w[p], wsem[p])
        pltpu.async_copy(idx_i_hbm.at[pl.ds(eb, CHUNK)], idxi[q], isem[q])

    def process(j, p, q, prefetch=True):
        eb = base + j * CHUNK
        pltpu.make_async_copy(
            h_hbm.at[idxj_all.at[pl.ds(j * CHUNK, CHUNK)]], rows[p], gsem[p]
        ).wait()
        pltpu.make_async_copy(
            wij_hbm.at[pl.ds(eb, CHUNK)], w[p], wsem[p]).wait()
        pltpu.make_async_copy(
            idx_i_hbm.at[pl.ds(eb, CHUNK)], idxi[q], isem[q]).wait()

        @pl.when(jnp.asarray(j, jnp.int32) >= 2)
        def _():
            pltpu.make_async_copy(
                outb[p], agg_s.at[idxi[(q + 2) % 4]], ssem[p]).wait()

        rows_p, w_p, out_p = rows[p], w[p], outb[p]

        @plsc.parallel_loop(0, CHUNK, 1, unroll=4)
        def _(r):
            for g in range(D // 16):
                sl = pl.ds(g * 16, 16)
                out_p[r, sl] = rows_p[r, sl] * w_p[r, sl]

        pltpu.async_copy(out_p, agg_s.at[idxi[q]], ssem[p], add=True)

        if prefetch:
            @pl.when(jnp.asarray(j, jnp.int32) + 2 < NCHUNK)
            def _():
                issue_loads(j + 2, p, (q + 2) % 4)

    issue_loads(0, 0, 0)
    issue_loads(1, 1, 1)

    def quad(it, carry):
        for b in range(4):
            process(4 * it + b, b % 2, b)
        return carry

    lax.fori_loop(0, (NCHUNK - 2) // 4, quad, 0)
    process(NCHUNK - 2, 0, 0, prefetch=False)
    process(NCHUNK - 1, 1, 1, prefetch=False)

    pltpu.make_async_copy(outb[0], agg_s.at[idxi[0]], ssem[0]).wait()
    pltpu.make_async_copy(outb[1], agg_s.at[idxi[1]], ssem[1]).wait()

    plsc.subcore_barrier()
    pltpu.sync_copy(
        agg_s.at[pl.ds(s * RPS, RPS)],
        out_hbm.at[c, pl.ds(s * RPS, RPS)],
    )


def _sc_conv(h, wij, idx_i, idx_j):
    mesh = plsc.VectorSubcoreMesh(core_axis_name="c", subcore_axis_name="s")
    run = pl.kernel(
        _sc_conv_body,
        out_type=jax.ShapeDtypeStruct((NC, P, D), jnp.float32),
        mesh=mesh,
        scratch_types=[
            pltpu.VMEM((EPW,), jnp.int32),
            pltpu.VMEM((CHUNK, D), jnp.float32),
            pltpu.VMEM((CHUNK, D), jnp.float32),
            pltpu.VMEM((CHUNK, D), jnp.float32),
            pltpu.VMEM((CHUNK, D), jnp.float32),
            pltpu.VMEM((CHUNK, D), jnp.float32),
            pltpu.VMEM((CHUNK, D), jnp.float32),
            pltpu.VMEM((CHUNK,), jnp.int32),
            pltpu.VMEM((CHUNK,), jnp.int32),
            pltpu.VMEM((CHUNK,), jnp.int32),
            pltpu.VMEM((CHUNK,), jnp.int32),
            pltpu.VMEM_SHARED((P, D), jnp.float32),
        ] + [pltpu.SemaphoreType.DMA] * 10,
    )
    return run(h, wij, idx_i, idx_j)


def _out_body(agg_ref, w1_ref, b1_ref, w2_ref, b2_ref, o_ref):
    a = agg_ref[0] + agg_ref[1]
    t = _ssp(
        jnp.dot(a, w1_ref[...], preferred_element_type=jnp.float32)
        + b1_ref[...]
    )
    o_ref[...] = (
        jnp.dot(t, w2_ref[...], preferred_element_type=jnp.float32)
        + b2_ref[...]
    )


def _compute_out(aggP, W1, b1, W2, b2):
    blk = 2000
    return pl.pallas_call(
        _out_body,
        grid=(N_ATOMS // blk,),
        in_specs=[
            pl.BlockSpec((NC, blk, D), lambda i: (0, i, 0)),
            pl.BlockSpec((D, D), lambda i: (0, 0)),
            pl.BlockSpec((1, D), lambda i: (0, 0)),
            pl.BlockSpec((D, D), lambda i: (0, 0)),
            pl.BlockSpec((1, D), lambda i: (0, 0)),
        ],
        out_specs=pl.BlockSpec((blk, D), lambda i: (i, 0)),
        out_shape=jax.ShapeDtypeStruct((N_ATOMS, D), jnp.float32),
    )(aggP, W1, b1.reshape(1, D), W2, b2.reshape(1, D))


@jax.jit
def kernel(x, f_ij, idx_i, idx_j, rcut_ij,
           W_in2f, b_in2f, W_f1, b_f1, W_f2, b_f2,
           W_o1, b_o1, W_o2, b_o2):
    idx_i = idx_i.astype(jnp.int32)
    idx_j = idx_j.astype(jnp.int32)
    h = _compute_h(x, W_in2f, b_in2f)
    wij = _compute_wij(f_ij, rcut_ij, W_f1, b_f1, W_f2, b_f2)
    aggP = _sc_conv(h, wij, idx_i, idx_j)
    return _compute_out(aggP, W_o1, b_o1, W_o2, b_o2)


# speedup vs baseline: 3.1787x; 1.1192x over previous
import functools

import jax
import jax.numpy as jnp
from jax import lax
from jax.experimental import pallas as pl
from jax.experimental.pallas import tpu as pltpu
from jax.experimental.pallas import tpu_sc as plsc

N_ATOMS = 10000
N_EDGES = 320000
D = 128
N_RBF = 20

_LN2 = 0.6931471805599453

NC, NS = 2, 16
NW = NC * NS
EPW = N_EDGES // NW
CHUNK = 40
NCHUNK = EPW // CHUNK
P = 10240
RPS = P // NS


def _ssp(v):
    return jnp.maximum(v, 0.0) + jnp.log1p(jnp.exp(-jnp.abs(v))) - _LN2


def _h_body(x_ref, w_ref, b_ref, o_ref):
    o_ref[...] = (
        jnp.dot(x_ref[...], w_ref[...], preferred_element_type=jnp.float32)
        + b_ref[...]
    )


def _compute_h(x, W, b):
    blk = 2000
    return pl.pallas_call(
        _h_body,
        grid=(N_ATOMS // blk,),
        in_specs=[
            pl.BlockSpec((blk, D), lambda i: (i, 0)),
            pl.BlockSpec((D, D), lambda i: (0, 0)),
            pl.BlockSpec((1, D), lambda i: (0, 0)),
        ],
        out_specs=pl.BlockSpec((blk, D), lambda i: (i, 0)),
        out_shape=jax.ShapeDtypeStruct((N_ATOMS, D), jnp.float32),
    )(x, W, b.reshape(1, D))


def _wij_body(f_ref, w1_ref, b1_ref, w2_ref, b2_ref, o_ref):
    t = _ssp(
        jnp.dot(f_ref[...], w1_ref[...], preferred_element_type=jnp.float32)
        + b1_ref[...]
    )
    o_ref[...] = (
        jnp.dot(t, w2_ref[...], preferred_element_type=jnp.float32)
        + b2_ref[...]
    )


def _compute_wij(f_ij, W1, b1, W2, b2):
    blk = 2560
    return pl.pallas_call(
        _wij_body,
        grid=(N_EDGES // blk,),
        in_specs=[
            pl.BlockSpec((blk, N_RBF), lambda i: (i, 0)),
            pl.BlockSpec((N_RBF, D), lambda i: (0, 0)),
            pl.BlockSpec((1, D), lambda i: (0, 0)),
            pl.BlockSpec((D, D), lambda i: (0, 0)),
            pl.BlockSpec((1, D), lambda i: (0, 0)),
        ],
        out_specs=pl.BlockSpec((blk, D), lambda i: (i, 0)),
        out_shape=jax.ShapeDtypeStruct((N_EDGES, D), jnp.float32),
    )(f_ij, W1, b1.reshape(1, D), W2, b2.reshape(1, D))


def _sc_conv_body(h_hbm, wij_hbm, rc_hbm, idx_i_hbm, idx_j_hbm, out_hbm,
                  idxj_all,
                  rows0, rows1, w0, w1, out0, out1,
                  rc0, rc1,
                  idxi0, idxi1, idxi2, idxi3,
                  agg_s,
                  gsem0, gsem1, wsem0, wsem1, rsem0, rsem1,
                  isem0, isem1, isem2, isem3, ssem0, ssem1):
    c = lax.axis_index("c")
    s = lax.axis_index("s")
    wid = s * NC + c
    base = wid * EPW

    rows = (rows0, rows1)
    w = (w0, w1)
    outb = (out0, out1)
    rc = (rc0, rc1)
    idxi = (idxi0, idxi1, idxi2, idxi3)
    gsem = (gsem0, gsem1)
    wsem = (wsem0, wsem1)
    rsem = (rsem0, rsem1)
    isem = (isem0, isem1, isem2, isem3)
    ssem = (ssem0, ssem1)

    zero = jnp.zeros((16,), jnp.float32)

    @plsc.parallel_loop(0, CHUNK, 1, unroll=4)
    def _(r):
        for g in range(D // 16):
            out0[r, pl.ds(g * 16, 16)] = zero

    for k in range(RPS // CHUNK):
        pltpu.sync_copy(out0, agg_s.at[pl.ds(s * RPS + k * CHUNK, CHUNK)])
    plsc.subcore_barrier()

    pltpu.sync_copy(idx_j_hbm.at[pl.ds(base, EPW)], idxj_all)

    def issue_loads(j, p, q):
        eb = base + j * CHUNK
        pltpu.async_copy(
            h_hbm.at[idxj_all.at[pl.ds(j * CHUNK, CHUNK)]], rows[p], gsem[p])
        pltpu.async_copy(wij_hbm.at[pl.ds(eb, CHUNK)], w[p], wsem[p])
        pltpu.async_copy(rc_hbm.at[pl.ds(eb, CHUNK)], rc[p], rsem[p])
        pltpu.async_copy(idx_i_hbm.at[pl.ds(eb, CHUNK)], idxi[q], isem[q])

    def process(j, p, q, prefetch=True):
        eb = base + j * CHUNK
        pltpu.make_async_copy(
            h_hbm.at[idxj_all.at[pl.ds(j * CHUNK, CHUNK)]], rows[p], gsem[p]
        ).wait()
        pltpu.make_async_copy(
            wij_hbm.at[pl.ds(eb, CHUNK)], w[p], wsem[p]).wait()
        pltpu.make_async_copy(
            rc_hbm.at[pl.ds(eb, CHUNK)], rc[p], rsem[p]).wait()
        pltpu.make_async_copy(
            idx_i_hbm.at[pl.ds(eb, CHUNK)], idxi[q], isem[q]).wait()

        @pl.when(jnp.asarray(j, jnp.int32) >= 2)
        def _():
            pltpu.make_async_copy(
                outb[p], agg_s.at[idxi[(q + 2) % 4]], ssem[p]).wait()

        rows_p, w_p, out_p, rc_p = rows[p], w[p], outb[p], rc[p]

        @plsc.parallel_loop(0, CHUNK, 1, unroll=4)
        def _(r):
            scale = rc_p[pl.ds(r, 1)][0]
            for g in range(D // 16):
                sl = pl.ds(g * 16, 16)
                out_p[r, sl] = rows_p[r, sl] * w_p[r, sl] * scale

        pltpu.async_copy(out_p, agg_s.at[idxi[q]], ssem[p], add=True)

        if prefetch:
            @pl.when(jnp.asarray(j, jnp.int32) + 2 < NCHUNK)
            def _():
                issue_loads(j + 2, p, (q + 2) % 4)

    issue_loads(0, 0, 0)
    issue_loads(1, 1, 1)

    def quad(it, carry):
        for b in range(4):
            process(4 * it + b, b % 2, b)
        return carry

    lax.fori_loop(0, (NCHUNK - 2) // 4, quad, 0)
    process(NCHUNK - 2, 0, 0, prefetch=False)
    process(NCHUNK - 1, 1, 1, prefetch=False)

    pltpu.make_async_copy(outb[0], agg_s.at[idxi[0]], ssem[0]).wait()
    pltpu.make_async_copy(outb[1], agg_s.at[idxi[1]], ssem[1]).wait()

    plsc.subcore_barrier()
    pltpu.sync_copy(
        agg_s.at[pl.ds(s * RPS, RPS)],
        out_hbm.at[c, pl.ds(s * RPS, RPS)],
    )


def _sc_conv(h, wij, rcut, idx_i, idx_j):
    mesh = plsc.VectorSubcoreMesh(core_axis_name="c", subcore_axis_name="s")
    run = pl.kernel(
        _sc_conv_body,
        out_type=jax.ShapeDtypeStruct((NC, P, D), jnp.float32),
        mesh=mesh,
        scratch_types=[
            pltpu.VMEM((EPW,), jnp.int32),
            pltpu.VMEM((CHUNK, D), jnp.float32),
            pltpu.VMEM((CHUNK, D), jnp.float32),
            pltpu.VMEM((CHUNK, D), jnp.float32),
            pltpu.VMEM((CHUNK, D), jnp.float32),
            pltpu.VMEM((CHUNK, D), jnp.float32),
            pltpu.VMEM((CHUNK, D), jnp.float32),
            pltpu.VMEM((CHUNK,), jnp.float32),
            pltpu.VMEM((CHUNK,), jnp.float32),
            pltpu.VMEM((CHUNK,), jnp.int32),
            pltpu.VMEM((CHUNK,), jnp.int32),
            pltpu.VMEM((CHUNK,), jnp.int32),
            pltpu.VMEM((CHUNK,), jnp.int32),
            pltpu.VMEM_SHARED((P, D), jnp.float32),
        ] + [pltpu.SemaphoreType.DMA] * 12,
    )
    return run(h, wij, rcut, idx_i, idx_j)


def _out_body(agg_ref, w1_ref, b1_ref, w2_ref, b2_ref, o_ref):
    a = agg_ref[0] + agg_ref[1]
    t = _ssp(
        jnp.dot(a, w1_ref[...], preferred_element_type=jnp.float32)
        + b1_ref[...]
    )
    o_ref[...] = (
        jnp.dot(t, w2_ref[...], preferred_element_type=jnp.float32)
        + b2_ref[...]
    )


def _compute_out(aggP, W1, b1, W2, b2):
    blk = 2000
    return pl.pallas_call(
        _out_body,
        grid=(N_ATOMS // blk,),
        in_specs=[
            pl.BlockSpec((NC, blk, D), lambda i: (0, i, 0)),
            pl.BlockSpec((D, D), lambda i: (0, 0)),
            pl.BlockSpec((1, D), lambda i: (0, 0)),
            pl.BlockSpec((D, D), lambda i: (0, 0)),
            pl.BlockSpec((1, D), lambda i: (0, 0)),
        ],
        out_specs=pl.BlockSpec((blk, D), lambda i: (i, 0)),
        out_shape=jax.ShapeDtypeStruct((N_ATOMS, D), jnp.float32),
    )(aggP, W1, b1.reshape(1, D), W2, b2.reshape(1, D))


@jax.jit
def kernel(x, f_ij, idx_i, idx_j, rcut_ij,
           W_in2f, b_in2f, W_f1, b_f1, W_f2, b_f2,
           W_o1, b_o1, W_o2, b_o2):
    idx_i = idx_i.astype(jnp.int32)
    idx_j = idx_j.astype(jnp.int32)
    h = _compute_h(x, W_in2f, b_in2f)
    wij = _compute_wij(f_ij, W_f1, b_f1, W_f2, b_f2)
    aggP = _sc_conv(h, wij, rcut_ij, idx_i, idx_j)
    return _compute_out(aggP, W_o1, b_o1, W_o2, b_o2)


# speedup vs baseline: 3.6767x; 1.1567x over previous
import functools

import jax
import jax.numpy as jnp
from jax import lax
from jax.experimental import pallas as pl
from jax.experimental.pallas import tpu as pltpu
from jax.experimental.pallas import tpu_sc as plsc

N_ATOMS = 10000
N_EDGES = 320000
D = 128
N_RBF = 20

_LN2 = 0.6931471805599453

NC, NS = 2, 16
NW = NC * NS
EPW = N_EDGES // NW
CHUNK = 40
NCHUNK = EPW // CHUNK
P = 10240
RPS = P // NS


def _ssp(v):
    return jnp.maximum(v, 0.0) + jnp.log1p(jnp.exp(-jnp.abs(v))) - _LN2


def _h_body(x_ref, w_ref, b_ref, o_ref):
    o_ref[...] = (
        jnp.dot(x_ref[...], w_ref[...], preferred_element_type=jnp.float32)
        + b_ref[...]
    )


def _compute_h(x, W, b):
    blk = 2000
    return pl.pallas_call(
        _h_body,
        grid=(N_ATOMS // blk,),
        in_specs=[
            pl.BlockSpec((blk, D), lambda i: (i, 0)),
            pl.BlockSpec((D, D), lambda i: (0, 0)),
            pl.BlockSpec((1, D), lambda i: (0, 0)),
        ],
        out_specs=pl.BlockSpec((blk, D), lambda i: (i, 0)),
        out_shape=jax.ShapeDtypeStruct((N_ATOMS, D), jnp.float32),
    )(x, W, b.reshape(1, D))


def _wij_body(f_ref, rct_ref, w1_ref, b1_ref, w2_ref, b2_ref, o_ref):
    pre = jnp.dot(
        f_ref[...], w1_ref[...], preferred_element_type=jnp.float32)
    t = _ssp(pre + b1_ref[...])
    wij = (
        jnp.dot(t, w2_ref[...], preferred_element_type=jnp.float32)
        + b2_ref[...]
    )
    blk = wij.shape[0]
    ncol = blk // 128
    tile = rct_ref[0]
    for c in range(ncol):
        o_ref[pl.ds(c * 128, 128), :] = (
            wij[c * 128:(c + 1) * 128, :] * tile[:, c:c + 1]
        )


def _compute_wij(f_ij, rct, W1, b1, W2, b2):
    blk = 2560
    return pl.pallas_call(
        _wij_body,
        grid=(N_EDGES // blk,),
        in_specs=[
            pl.BlockSpec((blk, N_RBF), lambda i: (i, 0)),
            pl.BlockSpec((1, D, blk // D), lambda i: (i, 0, 0)),
            pl.BlockSpec((N_RBF, D), lambda i: (0, 0)),
            pl.BlockSpec((1, D), lambda i: (0, 0)),
            pl.BlockSpec((D, D), lambda i: (0, 0)),
            pl.BlockSpec((1, D), lambda i: (0, 0)),
        ],
        out_specs=pl.BlockSpec((blk, D), lambda i: (i, 0)),
        out_shape=jax.ShapeDtypeStruct((N_EDGES, D), jnp.float32),
    )(f_ij, rct, W1, b1.reshape(1, D), W2, b2.reshape(1, D))


def _sc_conv_body(h_hbm, wij_hbm, idx_i_hbm, idx_j_hbm, out_hbm,
                  idxj_all,
                  rows0, rows1, w0, w1, out0, out1,
                  idxi0, idxi1, idxi2, idxi3,
                  agg_s,
                  gsem0, gsem1, wsem0, wsem1,
                  isem0, isem1, isem2, isem3, ssem0, ssem1):
    c = lax.axis_index("c")
    s = lax.axis_index("s")
    wid = s * NC + c
    base = wid * EPW

    rows = (rows0, rows1)
    w = (w0, w1)
    outb = (out0, out1)
    idxi = (idxi0, idxi1, idxi2, idxi3)
    gsem = (gsem0, gsem1)
    wsem = (wsem0, wsem1)
    isem = (isem0, isem1, isem2, isem3)
    ssem = (ssem0, ssem1)

    zero = jnp.zeros((16,), jnp.float32)

    @plsc.parallel_loop(0, CHUNK, 1, unroll=4)
    def _(r):
        for g in range(D // 16):
            out0[r, pl.ds(g * 16, 16)] = zero

    for k in range(RPS // CHUNK):
        pltpu.sync_copy(out0, agg_s.at[pl.ds(s * RPS + k * CHUNK, CHUNK)])
    plsc.subcore_barrier()

    pltpu.sync_copy(idx_j_hbm.at[pl.ds(base, EPW)], idxj_all)

    def issue_loads(j, p, q):
        eb = base + j * CHUNK
        pltpu.async_copy(
            h_hbm.at[idxj_all.at[pl.ds(j * CHUNK, CHUNK)]], rows[p], gsem[p])
        pltpu.async_copy(wij_hbm.at[pl.ds(eb, CHUNK)], w[p], wsem[p])
        pltpu.async_copy(idx_i_hbm.at[pl.ds(eb, CHUNK)], idxi[q], isem[q])

    def process(j, p, q, prefetch=True):
        eb = base + j * CHUNK
        pltpu.make_async_copy(
            h_hbm.at[idxj_all.at[pl.ds(j * CHUNK, CHUNK)]], rows[p], gsem[p]
        ).wait()
        pltpu.make_async_copy(
            wij_hbm.at[pl.ds(eb, CHUNK)], w[p], wsem[p]).wait()
        pltpu.make_async_copy(
            idx_i_hbm.at[pl.ds(eb, CHUNK)], idxi[q], isem[q]).wait()

        @pl.when(jnp.asarray(j, jnp.int32) >= 2)
        def _():
            pltpu.make_async_copy(
                outb[p], agg_s.at[idxi[(q + 2) % 4]], ssem[p]).wait()

        rows_p, w_p, out_p = rows[p], w[p], outb[p]

        @plsc.parallel_loop(0, CHUNK, 1, unroll=4)
        def _(r):
            for g in range(D // 16):
                sl = pl.ds(g * 16, 16)
                out_p[r, sl] = rows_p[r, sl] * w_p[r, sl]

        pltpu.async_copy(out_p, agg_s.at[idxi[q]], ssem[p], add=True)

        if prefetch:
            @pl.when(jnp.asarray(j, jnp.int32) + 2 < NCHUNK)
            def _():
                issue_loads(j + 2, p, (q + 2) % 4)

    issue_loads(0, 0, 0)
    issue_loads(1, 1, 1)

    def quad(it, carry):
        for b in range(4):
            process(4 * it + b, b % 2, b)
        return carry

    lax.fori_loop(0, (NCHUNK - 2) // 4, quad, 0)
    process(NCHUNK - 2, 0, 0, prefetch=False)
    process(NCHUNK - 1, 1, 1, prefetch=False)

    pltpu.make_async_copy(outb[0], agg_s.at[idxi[0]], ssem[0]).wait()
    pltpu.make_async_copy(outb[1], agg_s.at[idxi[1]], ssem[1]).wait()

    plsc.subcore_barrier()
    pltpu.sync_copy(
        agg_s.at[pl.ds(s * RPS, RPS)],
        out_hbm.at[c, pl.ds(s * RPS, RPS)],
    )


def _sc_conv(h, wij, idx_i, idx_j):
    mesh = plsc.VectorSubcoreMesh(core_axis_name="c", subcore_axis_name="s")
    run = pl.kernel(
        _sc_conv_body,
        out_type=jax.ShapeDtypeStruct((NC, P, D), jnp.float32),
        mesh=mesh,
        scratch_types=[
            pltpu.VMEM((EPW,), jnp.int32),
            pltpu.VMEM((CHUNK, D), jnp.float32),
            pltpu.VMEM((CHUNK, D), jnp.float32),
            pltpu.VMEM((CHUNK, D), jnp.float32),
            pltpu.VMEM((CHUNK, D), jnp.float32),
            pltpu.VMEM((CHUNK, D), jnp.float32),
            pltpu.VMEM((CHUNK, D), jnp.float32),
            pltpu.VMEM((CHUNK,), jnp.int32),
            pltpu.VMEM((CHUNK,), jnp.int32),
            pltpu.VMEM((CHUNK,), jnp.int32),
            pltpu.VMEM((CHUNK,), jnp.int32),
            pltpu.VMEM_SHARED((P, D), jnp.float32),
        ] + [pltpu.SemaphoreType.DMA] * 10,
    )
    return run(h, wij, idx_i, idx_j)


def _out_body(agg_ref, w1_ref, b1_ref, w2_ref, b2_ref, o_ref):
    a = agg_ref[0] + agg_ref[1]
    t = _ssp(
        jnp.dot(a, w1_ref[...], preferred_element_type=jnp.float32)
        + b1_ref[...]
    )
    o_ref[...] = (
        jnp.dot(t, w2_ref[...], preferred_element_type=jnp.float32)
        + b2_ref[...]
    )


def _compute_out(aggP, W1, b1, W2, b2):
    blk = 2000
    return pl.pallas_call(
        _out_body,
        grid=(N_ATOMS // blk,),
        in_specs=[
            pl.BlockSpec((NC, blk, D), lambda i: (0, i, 0)),
            pl.BlockSpec((D, D), lambda i: (0, 0)),
            pl.BlockSpec((1, D), lambda i: (0, 0)),
            pl.BlockSpec((D, D), lambda i: (0, 0)),
            pl.BlockSpec((1, D), lambda i: (0, 0)),
        ],
        out_specs=pl.BlockSpec((blk, D), lambda i: (i, 0)),
        out_shape=jax.ShapeDtypeStruct((N_ATOMS, D), jnp.float32),
    )(aggP, W1, b1.reshape(1, D), W2, b2.reshape(1, D))


@jax.jit
def kernel(x, f_ij, idx_i, idx_j, rcut_ij,
           W_in2f, b_in2f, W_f1, b_f1, W_f2, b_f2,
           W_o1, b_o1, W_o2, b_o2):
    idx_i = idx_i.astype(jnp.int32)
    idx_j = idx_j.astype(jnp.int32)
    h = _compute_h(x, W_in2f, b_in2f)
    blk = 2560
    rct = jnp.transpose(
        rcut_ij.reshape(N_EDGES // blk, blk // D, D), (0, 2, 1))
    wij = _compute_wij(f_ij, rct, W_f1, b_f1, W_f2, b_f2)
    aggP = _sc_conv(h, wij, idx_i, idx_j)
    return _compute_out(aggP, W_o1, b_o1, W_o2, b_o2)


# speedup vs baseline: 4.1521x; 1.1293x over previous
import functools

import jax
import jax.numpy as jnp
from jax import lax
from jax.experimental import pallas as pl
from jax.experimental.pallas import tpu as pltpu
from jax.experimental.pallas import tpu_sc as plsc

N_ATOMS = 10000
N_EDGES = 320000
D = 128
N_RBF = 20

_LN2 = 0.6931471805599453

NC, NS = 2, 16
NW = NC * NS
EPW = N_EDGES // NW
CHUNK = 40
NCHUNK = EPW // CHUNK
P = 10240
RPS = P // NS


def _ssp(v):
    return jnp.maximum(v, 0.0) + jnp.log1p(jnp.exp(-jnp.abs(v))) - _LN2


def _h_body(x_ref, w_ref, b_ref, o_ref):
    o_ref[...] = (
        jnp.dot(x_ref[...], w_ref[...], preferred_element_type=jnp.float32)
        + b_ref[...]
    )


def _compute_h(x, W, b):
    blk = 2000
    return pl.pallas_call(
        _h_body,
        grid=(N_ATOMS // blk,),
        in_specs=[
            pl.BlockSpec((blk, D), lambda i: (i, 0)),
            pl.BlockSpec((D, D), lambda i: (0, 0)),
            pl.BlockSpec((1, D), lambda i: (0, 0)),
        ],
        out_specs=pl.BlockSpec((blk, D), lambda i: (i, 0)),
        out_shape=jax.ShapeDtypeStruct((N_ATOMS, D), jnp.float32),
    )(x, W, b.reshape(1, D))


def _wij_body(f_ref, rct_ref, w1_ref, b1_ref, w2_ref, b2_ref, o_ref):
    pre = jnp.dot(
        f_ref[...], w1_ref[...], preferred_element_type=jnp.float32)
    t = _ssp(pre + b1_ref[...])
    wij = (
        jnp.dot(t, w2_ref[...], preferred_element_type=jnp.float32)
        + b2_ref[...]
    )
    blk = wij.shape[0]
    ncol = blk // 128
    tile = rct_ref[0]
    for c in range(ncol):
        o_ref[pl.ds(c * 128, 128), :] = (
            wij[c * 128:(c + 1) * 128, :] * tile[:, c:c + 1]
        )


def _compute_wij(f_ij, rct, W1, b1, W2, b2, blk, nblk, base_blk):
    return pl.pallas_call(
        _wij_body,
        grid=(nblk,),
        in_specs=[
            pl.BlockSpec((blk, N_RBF), lambda i, b=base_blk: (i + b, 0)),
            pl.BlockSpec(
                (1, D, blk // D), lambda i, b=base_blk: (i + b, 0, 0)),
            pl.BlockSpec((N_RBF, D), lambda i: (0, 0)),
            pl.BlockSpec((1, D), lambda i: (0, 0)),
            pl.BlockSpec((D, D), lambda i: (0, 0)),
            pl.BlockSpec((1, D), lambda i: (0, 0)),
        ],
        out_specs=pl.BlockSpec((blk, D), lambda i: (i, 0)),
        out_shape=jax.ShapeDtypeStruct((nblk * blk, D), jnp.float32),
    )(f_ij, rct, W1, b1.reshape(1, D), W2, b2.reshape(1, D))


def _sc_conv_body(epw, ebase,
                  h_hbm, wij_hbm, idx_i_hbm, idx_j_hbm, out_hbm,
                  idxj_all,
                  rows0, rows1, w0, w1, out0, out1,
                  idxi0, idxi1, idxi2, idxi3,
                  agg_s,
                  gsem0, gsem1, wsem0, wsem1,
                  isem0, isem1, isem2, isem3, ssem0, ssem1):
    nchunk = epw // CHUNK
    c = lax.axis_index("c")
    s = lax.axis_index("s")
    wid = s * NC + c
    base = wid * epw
    ibase = ebase + base

    rows = (rows0, rows1)
    w = (w0, w1)
    outb = (out0, out1)
    idxi = (idxi0, idxi1, idxi2, idxi3)
    gsem = (gsem0, gsem1)
    wsem = (wsem0, wsem1)
    isem = (isem0, isem1, isem2, isem3)
    ssem = (ssem0, ssem1)

    zero = jnp.zeros((16,), jnp.float32)

    @plsc.parallel_loop(0, CHUNK, 1, unroll=4)
    def _(r):
        for g in range(D // 16):
            out0[r, pl.ds(g * 16, 16)] = zero

    for k in range(RPS // CHUNK):
        pltpu.sync_copy(out0, agg_s.at[pl.ds(s * RPS + k * CHUNK, CHUNK)])
    plsc.subcore_barrier()

    pltpu.sync_copy(idx_j_hbm.at[pl.ds(ibase, epw)], idxj_all)

    def issue_loads(j, p, q):
        eb = base + j * CHUNK
        pltpu.async_copy(
            h_hbm.at[idxj_all.at[pl.ds(j * CHUNK, CHUNK)]], rows[p], gsem[p])
        pltpu.async_copy(wij_hbm.at[pl.ds(eb, CHUNK)], w[p], wsem[p])
        pltpu.async_copy(
            idx_i_hbm.at[pl.ds(ebase + eb, CHUNK)], idxi[q], isem[q])

    def process(j, p, q, prefetch=True):
        eb = base + j * CHUNK
        pltpu.make_async_copy(
            h_hbm.at[idxj_all.at[pl.ds(j * CHUNK, CHUNK)]], rows[p], gsem[p]
        ).wait()
        pltpu.make_async_copy(
            wij_hbm.at[pl.ds(eb, CHUNK)], w[p], wsem[p]).wait()
        pltpu.make_async_copy(
            idx_i_hbm.at[pl.ds(ebase + eb, CHUNK)], idxi[q], isem[q]).wait()

        @pl.when(jnp.asarray(j, jnp.int32) >= 2)
        def _():
            pltpu.make_async_copy(
                outb[p], agg_s.at[idxi[(q + 2) % 4]], ssem[p]).wait()

        rows_p, w_p, out_p = rows[p], w[p], outb[p]

        @plsc.parallel_loop(0, CHUNK, 1, unroll=4)
        def _(r):
            for g in range(D // 16):
                sl = pl.ds(g * 16, 16)
                out_p[r, sl] = rows_p[r, sl] * w_p[r, sl]

        pltpu.async_copy(out_p, agg_s.at[idxi[q]], ssem[p], add=True)

        if prefetch:
            @pl.when(jnp.asarray(j, jnp.int32) + 2 < nchunk)
            def _():
                issue_loads(j + 2, p, (q + 2) % 4)

    issue_loads(0, 0, 0)
    issue_loads(1, 1, 1)

    def quad(it, carry):
        for b in range(4):
            process(4 * it + b, b % 2, b)
        return carry

    nquad = nchunk // 4
    lax.fori_loop(0, nquad, quad, 0)
    for j in range(4 * nquad, nchunk):
        process(j, j % 2, j % 4, prefetch=False)

    for j in (nchunk - 2, nchunk - 1):
        pltpu.make_async_copy(
            outb[j % 2], agg_s.at[idxi[j % 4]], ssem[j % 2]).wait()

    plsc.subcore_barrier()
    pltpu.sync_copy(
        agg_s.at[pl.ds(s * RPS, RPS)],
        out_hbm.at[c, pl.ds(s * RPS, RPS)],
    )


def _sc_conv(h, wij, idx_i, idx_j, n_edges, ebase):
    epw = n_edges // NW
    mesh = plsc.VectorSubcoreMesh(core_axis_name="c", subcore_axis_name="s")
    run = pl.kernel(
        functools.partial(_sc_conv_body, epw, ebase),
        out_type=jax.ShapeDtypeStruct((NC, P, D), jnp.float32),
        mesh=mesh,
        scratch_types=[
            pltpu.VMEM((epw,), jnp.int32),
            pltpu.VMEM((CHUNK, D), jnp.float32),
            pltpu.VMEM((CHUNK, D), jnp.float32),
            pltpu.VMEM((CHUNK, D), jnp.float32),
            pltpu.VMEM((CHUNK, D), jnp.float32),
            pltpu.VMEM((CHUNK, D), jnp.float32),
            pltpu.VMEM((CHUNK, D), jnp.float32),
            pltpu.VMEM((CHUNK,), jnp.int32),
            pltpu.VMEM((CHUNK,), jnp.int32),
            pltpu.VMEM((CHUNK,), jnp.int32),
            pltpu.VMEM((CHUNK,), jnp.int32),
            pltpu.VMEM_SHARED((P, D), jnp.float32),
        ] + [pltpu.SemaphoreType.DMA] * 10,
    )
    return run(h, wij, idx_i, idx_j)


def _out_body(agg_a_ref, agg_b_ref, w1_ref, b1_ref, w2_ref, b2_ref, o_ref):
    a = (agg_a_ref[0] + agg_a_ref[1]) + (agg_b_ref[0] + agg_b_ref[1])
    t = _ssp(
        jnp.dot(a, w1_ref[...], preferred_element_type=jnp.float32)
        + b1_ref[...]
    )
    o_ref[...] = (
        jnp.dot(t, w2_ref[...], preferred_element_type=jnp.float32)
        + b2_ref[...]
    )


def _compute_out(aggA, aggB, W1, b1, W2, b2):
    blk = 2000
    return pl.pallas_call(
        _out_body,
        grid=(N_ATOMS // blk,),
        in_specs=[
            pl.BlockSpec((NC, blk, D), lambda i: (0, i, 0)),
            pl.BlockSpec((NC, blk, D), lambda i: (0, i, 0)),
            pl.BlockSpec((D, D), lambda i: (0, 0)),
            pl.BlockSpec((1, D), lambda i: (0, 0)),
            pl.BlockSpec((D, D), lambda i: (0, 0)),
            pl.BlockSpec((1, D), lambda i: (0, 0)),
        ],
        out_specs=pl.BlockSpec((blk, D), lambda i: (i, 0)),
        out_shape=jax.ShapeDtypeStruct((N_ATOMS, D), jnp.float32),
    )(aggA, aggB, W1, b1.reshape(1, D), W2, b2.reshape(1, D))


@jax.jit
def kernel(x, f_ij, idx_i, idx_j, rcut_ij,
           W_in2f, b_in2f, W_f1, b_f1, W_f2, b_f2,
           W_o1, b_o1, W_o2, b_o2):
    idx_i = idx_i.astype(jnp.int32)
    idx_j = idx_j.astype(jnp.int32)
    h = _compute_h(x, W_in2f, b_in2f)
    blk = 3200
    nblk = N_EDGES // blk
    half = N_EDGES // 2
    rct = jnp.transpose(
        rcut_ij.reshape(nblk, blk // D, D), (0, 2, 1))
    wij_a = _compute_wij(f_ij, rct, W_f1, b_f1, W_f2, b_f2,
                         blk, nblk // 2, 0)
    wij_b = _compute_wij(f_ij, rct, W_f1, b_f1, W_f2, b_f2,
                         blk, nblk // 2, nblk // 2)
    agg_a = _sc_conv(h, wij_a, idx_i, idx_j, half, 0)
    agg_b = _sc_conv(h, wij_b, idx_i, idx_j, half, half)
    return _compute_out(agg_a, agg_b, W_o1, b_o1, W_o2, b_o2)


# speedup vs baseline: 4.2300x; 1.0188x over previous
import functools

import jax
import jax.numpy as jnp
from jax import lax
from jax.experimental import pallas as pl
from jax.experimental.pallas import tpu as pltpu
from jax.experimental.pallas import tpu_sc as plsc

N_ATOMS = 10000
N_EDGES = 320000
D = 128
N_RBF = 20

_LN2 = 0.6931471805599453

NC, NS = 2, 16
NW = NC * NS
EPW = N_EDGES // NW
CHUNK = 40
NCHUNK = EPW // CHUNK
P = 10240
RPS = P // NS


def _ssp(v):
    return jnp.maximum(v, 0.0) + jnp.log1p(jnp.exp(-jnp.abs(v))) - _LN2


def _h_body(x_ref, w_ref, b_ref, o_ref):
    o_ref[...] = (
        jnp.dot(x_ref[...], w_ref[...], preferred_element_type=jnp.float32)
        + b_ref[...]
    )


def _compute_h(x, W, b):
    blk = 2000
    return pl.pallas_call(
        _h_body,
        grid=(N_ATOMS // blk,),
        in_specs=[
            pl.BlockSpec((blk, D), lambda i: (i, 0)),
            pl.BlockSpec((D, D), lambda i: (0, 0)),
            pl.BlockSpec((1, D), lambda i: (0, 0)),
        ],
        out_specs=pl.BlockSpec((blk, D), lambda i: (i, 0)),
        out_shape=jax.ShapeDtypeStruct((N_ATOMS, D), jnp.float32),
    )(x, W, b.reshape(1, D))


def _wij_body(f_ref, rct_ref, w1_ref, b1_ref, w2_ref, b2_ref, o_ref):
    pre = jnp.dot(
        f_ref[...], w1_ref[...], preferred_element_type=jnp.float32)
    t = _ssp(pre + b1_ref[...])
    wij = (
        jnp.dot(t, w2_ref[...], preferred_element_type=jnp.float32)
        + b2_ref[...]
    )
    blk = wij.shape[0]
    ncol = blk // 128
    tile = rct_ref[0]
    for c in range(ncol):
        o_ref[pl.ds(c * 128, 128), :] = (
            wij[c * 128:(c + 1) * 128, :] * tile[:, c:c + 1]
        )


def _compute_wij(f_half, rct, W1, b1, W2, b2, blk, nblk, base_blk):
    return pl.pallas_call(
        _wij_body,
        grid=(nblk,),
        in_specs=[
            pl.BlockSpec((blk, N_RBF), lambda i: (i, 0)),
            pl.BlockSpec(
                (1, D, blk // D), lambda i, b=base_blk: (i + b, 0, 0)),
            pl.BlockSpec((N_RBF, D), lambda i: (0, 0)),
            pl.BlockSpec((1, D), lambda i: (0, 0)),
            pl.BlockSpec((D, D), lambda i: (0, 0)),
            pl.BlockSpec((1, D), lambda i: (0, 0)),
        ],
        out_specs=pl.BlockSpec((blk, D), lambda i: (i, 0)),
        out_shape=jax.ShapeDtypeStruct((nblk * blk, D), jnp.float32),
    )(f_half, rct, W1, b1.reshape(1, D), W2, b2.reshape(1, D))


def _sc_conv_body(epw, ebase,
                  h_hbm, wij_hbm, idx_i_hbm, idx_j_hbm, out_hbm,
                  idxj_all,
                  rows0, rows1, w0, w1, out0, out1,
                  idxi0, idxi1, idxi2, idxi3,
                  agg_s,
                  gsem0, gsem1, wsem0, wsem1,
                  isem0, isem1, isem2, isem3, ssem0, ssem1):
    nchunk = epw // CHUNK
    c = lax.axis_index("c")
    s = lax.axis_index("s")
    wid = s * NC + c
    base = wid * epw
    ibase = ebase + base

    rows = (rows0, rows1)
    w = (w0, w1)
    outb = (out0, out1)
    idxi = (idxi0, idxi1, idxi2, idxi3)
    gsem = (gsem0, gsem1)
    wsem = (wsem0, wsem1)
    isem = (isem0, isem1, isem2, isem3)
    ssem = (ssem0, ssem1)

    zero = jnp.zeros((16,), jnp.float32)

    @plsc.parallel_loop(0, CHUNK, 1, unroll=4)
    def _(r):
        for g in range(D // 16):
            out0[r, pl.ds(g * 16, 16)] = zero

    for k in range(RPS // CHUNK):
        pltpu.sync_copy(out0, agg_s.at[pl.ds(s * RPS + k * CHUNK, CHUNK)])
    plsc.subcore_barrier()

    pltpu.sync_copy(idx_j_hbm.at[pl.ds(ibase, epw)], idxj_all)

    def issue_loads(j, p, q):
        eb = base + j * CHUNK
        pltpu.async_copy(
            h_hbm.at[idxj_all.at[pl.ds(j * CHUNK, CHUNK)]], rows[p], gsem[p])
        pltpu.async_copy(wij_hbm.at[pl.ds(eb, CHUNK)], w[p], wsem[p])
        pltpu.async_copy(
            idx_i_hbm.at[pl.ds(ebase + eb, CHUNK)], idxi[q], isem[q])

    def process(j, p, q, prefetch=True):
        eb = base + j * CHUNK
        pltpu.make_async_copy(
            h_hbm.at[idxj_all.at[pl.ds(j * CHUNK, CHUNK)]], rows[p], gsem[p]
        ).wait()
        pltpu.make_async_copy(
            wij_hbm.at[pl.ds(eb, CHUNK)], w[p], wsem[p]).wait()
        pltpu.make_async_copy(
            idx_i_hbm.at[pl.ds(ebase + eb, CHUNK)], idxi[q], isem[q]).wait()

        @pl.when(jnp.asarray(j, jnp.int32) >= 2)
        def _():
            pltpu.make_async_copy(
                outb[p], agg_s.at[idxi[(q + 2) % 4]], ssem[p]).wait()

        rows_p, w_p, out_p = rows[p], w[p], outb[p]

        @plsc.parallel_loop(0, CHUNK, 1, unroll=4)
        def _(r):
            for g in range(D // 16):
                sl = pl.ds(g * 16, 16)
                out_p[r, sl] = rows_p[r, sl] * w_p[r, sl]

        pltpu.async_copy(out_p, agg_s.at[idxi[q]], ssem[p], add=True)

        if prefetch:
            @pl.when(jnp.asarray(j, jnp.int32) + 2 < nchunk)
            def _():
                issue_loads(j + 2, p, (q + 2) % 4)

    issue_loads(0, 0, 0)
    issue_loads(1, 1, 1)

    def quad(it, carry):
        for b in range(4):
            process(4 * it + b, b % 2, b)
        return carry

    nquad = nchunk // 4
    lax.fori_loop(0, nquad, quad, 0)
    for j in range(4 * nquad, nchunk):
        process(j, j % 2, j % 4, prefetch=False)

    for j in (nchunk - 2, nchunk - 1):
        pltpu.make_async_copy(
            outb[j % 2], agg_s.at[idxi[j % 4]], ssem[j % 2]).wait()

    plsc.subcore_barrier()
    pltpu.sync_copy(
        agg_s.at[pl.ds(s * RPS, RPS)],
        out_hbm.at[c, pl.ds(s * RPS, RPS)],
    )


def _sc_conv(h, wij, idx_i, idx_j, n_edges, ebase):
    epw = n_edges // NW
    mesh = plsc.VectorSubcoreMesh(core_axis_name="c", subcore_axis_name="s")
    run = pl.kernel(
        functools.partial(_sc_conv_body, epw, ebase),
        out_type=jax.ShapeDtypeStruct((NC, P, D), jnp.float32),
        mesh=mesh,
        scratch_types=[
            pltpu.VMEM((epw,), jnp.int32),
            pltpu.VMEM((CHUNK, D), jnp.float32),
            pltpu.VMEM((CHUNK, D), jnp.float32),
            pltpu.VMEM((CHUNK, D), jnp.float32),
            pltpu.VMEM((CHUNK, D), jnp.float32),
            pltpu.VMEM((CHUNK, D), jnp.float32),
            pltpu.VMEM((CHUNK, D), jnp.float32),
            pltpu.VMEM((CHUNK,), jnp.int32),
            pltpu.VMEM((CHUNK,), jnp.int32),
            pltpu.VMEM((CHUNK,), jnp.int32),
            pltpu.VMEM((CHUNK,), jnp.int32),
            pltpu.VMEM_SHARED((P, D), jnp.float32),
        ] + [pltpu.SemaphoreType.DMA] * 10,
    )
    return run(h, wij, idx_i, idx_j)


def _out_body(agg_a_ref, agg_b_ref, w1_ref, b1_ref, w2_ref, b2_ref, o_ref):
    a = (agg_a_ref[0] + agg_a_ref[1]) + (agg_b_ref[0] + agg_b_ref[1])
    t = _ssp(
        jnp.dot(a, w1_ref[...], preferred_element_type=jnp.float32)
        + b1_ref[...]
    )
    o_ref[...] = (
        jnp.dot(t, w2_ref[...], preferred_element_type=jnp.float32)
        + b2_ref[...]
    )


def _compute_out(aggA, aggB, W1, b1, W2, b2):
    blk = 2000
    return pl.pallas_call(
        _out_body,
        grid=(N_ATOMS // blk,),
        in_specs=[
            pl.BlockSpec((NC, blk, D), lambda i: (0, i, 0)),
            pl.BlockSpec((NC, blk, D), lambda i: (0, i, 0)),
            pl.BlockSpec((D, D), lambda i: (0, 0)),
            pl.BlockSpec((1, D), lambda i: (0, 0)),
            pl.BlockSpec((D, D), lambda i: (0, 0)),
            pl.BlockSpec((1, D), lambda i: (0, 0)),
        ],
        out_specs=pl.BlockSpec((blk, D), lambda i: (i, 0)),
        out_shape=jax.ShapeDtypeStruct((N_ATOMS, D), jnp.float32),
    )(aggA, aggB, W1, b1.reshape(1, D), W2, b2.reshape(1, D))


@jax.jit
def kernel(x, f_ij, idx_i, idx_j, rcut_ij,
           W_in2f, b_in2f, W_f1, b_f1, W_f2, b_f2,
           W_o1, b_o1, W_o2, b_o2):
    idx_i = idx_i.astype(jnp.int32)
    idx_j = idx_j.astype(jnp.int32)
    h = _compute_h(x, W_in2f, b_in2f)
    blk = 3200
    nblk = N_EDGES // blk
    half = N_EDGES // 2
    rct = jnp.transpose(
        rcut_ij.reshape(nblk, blk // D, D), (0, 2, 1))
    f_a = lax.slice(f_ij, (0, 0), (half, N_RBF))
    f_b = lax.slice(f_ij, (half, 0), (N_EDGES, N_RBF))
    wij_a = _compute_wij(f_a, rct, W_f1, b_f1, W_f2, b_f2,
                         blk, nblk // 2, 0)
    wij_b = _compute_wij(f_b, rct, W_f1, b_f1, W_f2, b_f2,
                         blk, nblk // 2, nblk // 2)
    agg_a = _sc_conv(h, wij_a, idx_i, idx_j, half, 0)
    agg_b = _sc_conv(h, wij_b, idx_i, idx_j, half, half)
    return _compute_out(agg_a, agg_b, W_o1, b_o1, W_o2, b_o2)
